# baseline (speedup 1.0000x reference)
import functools

import jax
import jax.numpy as jnp
from jax import lax
from jax.experimental import pallas as pl
from jax.experimental.pallas import tpu as pltpu
from jax.experimental.pallas import tpu_sc as plsc

B = 16384
F = 128
H1, H2, H3 = 1024, 512, 256

NC, NS = 2, 16
NW = NC * NS
CHUNK = 128
NSPLIT = 1
BSUB = B // NSPLIT
BPW = BSUB // NW
NCH = BPW // CHUNK

@functools.cache
def _make_sc_gather():
    mesh = plsc.VectorSubcoreMesh(core_axis_name="c", subcore_axis_name="s")

    @functools.partial(
        pl.kernel,
        mesh=mesh,
        out_type=jax.ShapeDtypeStruct((BSUB, 2 * F), jnp.float32),
        scratch_types=[
            pltpu.VMEM((NCH, CHUNK), jnp.int32),
            pltpu.VMEM((BPW, F), jnp.float32),
            pltpu.SemaphoreType.DMA,
        ],
    )
    def _sc_gather(uidx, midx, utab, mtab, x, idx_v, rows_v, sem):
        wid = lax.axis_index("s") * NC + lax.axis_index("c")
        base = wid * BPW
        pltpu.sync_copy(uidx.at[wid], idx_v)
        waits = [
            pltpu.async_copy(utab.at[idx_v.at[j]], rows_v.at[pl.ds(j * CHUNK, CHUNK)], sem)
            for j in range(NCH)
        ]
        for w in waits:
            w.wait()
        pltpu.sync_copy(rows_v, x.at[pl.ds(base, BPW), pl.ds(0, F)])
        pltpu.sync_copy(midx.at[wid], idx_v)
        waits = [
            pltpu.async_copy(mtab.at[idx_v.at[j]], rows_v.at[pl.ds(j * CHUNK, CHUNK)], sem)
            for j in range(NCH)
        ]
        for w in waits:
            w.wait()
        pltpu.sync_copy(rows_v, x.at[pl.ds(base, BPW), pl.ds(F, F)])

    return _sc_gather


BM = 1024


_NT = (((1,), (1,)), ((), ()))


def _dot_nt(a, w):
    return lax.dot_general(a, w, _NT, preferred_element_type=jnp.float32)


def _mlp_body(x, w1, b1, w2, b2, w3, b3, wf, bf, out):
    bf16 = jnp.bfloat16
    h = _dot_nt(x[...].astype(bf16), w1[...])
    h = jnp.maximum(h + b1[...], 0.0).astype(bf16)
    h = jnp.maximum(_dot_nt(h, w2[...]) + b2[...], 0.0).astype(bf16)
    h = jnp.maximum(_dot_nt(h, w3[...]) + b3[...], 0.0)
    z = jnp.sum(h * wf[...], axis=1) + bf[0, 0]
    out[...] = 4.5 * jax.nn.sigmoid(z) + 0.5


def _mlp(x, w1, b1, w2, b2, w3, b3, wf, bf, interpret=False):
    const = lambda i: (0, 0)
    return pl.pallas_call(
        _mlp_body,
        grid=(BSUB // BM,),
        in_specs=[
            pl.BlockSpec((BM, 2 * F), lambda i: (i, 0)),
            pl.BlockSpec((H1, 2 * F), const),
            pl.BlockSpec((1, H1), const),
            pl.BlockSpec((H2, H1), const),
            pl.BlockSpec((1, H2), const),
            pl.BlockSpec((H3, H2), const),
            pl.BlockSpec((1, H3), const),
            pl.BlockSpec((1, H3), const),
            pl.BlockSpec((1, 1), const),
        ],
        out_specs=pl.BlockSpec((BM,), lambda i: (i,)),
        out_shape=jax.ShapeDtypeStruct((BSUB,), jnp.float32),
        interpret=interpret,
    )(x, w1, b1, w2, b2, w3, b3, wf, bf)


def kernel(users, movies, user_table, movie_table, W1, b1, W2, b2, W3, b3, Wf, bf):
    uidx = users.reshape(NSPLIT, NW, NCH, CHUNK)
    midx = movies.reshape(NSPLIT, NW, NCH, CHUNK)
    w1 = W1.astype(jnp.bfloat16)
    w2 = W2.astype(jnp.bfloat16)
    w3 = W3.astype(jnp.bfloat16)
    b1r, b2r, b3r = b1.reshape(1, H1), b2.reshape(1, H2), b3.reshape(1, H3)
    bfr = bf.reshape(1, 1)
    gather = _make_sc_gather()
    outs = []
    for c in range(NSPLIT):
        x = gather(uidx[c], midx[c], user_table, movie_table)
        outs.append(_mlp(x, w1, b1r, w2, b2r, w3, b3r, Wf, bfr))
    return jnp.concatenate(outs, axis=0).reshape(B, 1)


# speedup vs baseline: 1.2179x; 1.2179x over previous
import functools

import jax
import jax.numpy as jnp
from jax import lax
from jax.experimental import pallas as pl
from jax.experimental.pallas import tpu as pltpu
from jax.experimental.pallas import tpu_sc as plsc

B = 16384
F = 128
H1, H2, H3 = 1024, 512, 256

NC, NS = 2, 16
NW = NC * NS
CHUNK = 128
NSPLIT = 1
BSUB = B // NSPLIT
BPW = BSUB // NW
NCH = BPW // CHUNK

@functools.cache
def _make_sc_gather():
    mesh = plsc.VectorSubcoreMesh(core_axis_name="c", subcore_axis_name="s")

    @functools.partial(
        pl.kernel,
        mesh=mesh,
        out_type=jax.ShapeDtypeStruct((BSUB, 2 * F), jnp.float32),
        scratch_types=[
            pltpu.VMEM((NCH, CHUNK), jnp.int32),
            pltpu.VMEM((BPW, F), jnp.float32),
            pltpu.SemaphoreType.DMA,
        ],
    )
    def _sc_gather(uidx, midx, utab, mtab, x, idx_v, rows_v, sem):
        wid = lax.axis_index("s") * NC + lax.axis_index("c")
        base = wid * BPW
        pltpu.sync_copy(uidx.at[wid], idx_v)
        waits = [
            pltpu.async_copy(utab.at[idx_v.at[j]], rows_v.at[pl.ds(j * CHUNK, CHUNK)], sem)
            for j in range(NCH)
        ]
        for w in waits:
            w.wait()
        pltpu.sync_copy(rows_v, x.at[pl.ds(base, BPW), pl.ds(0, F)])
        pltpu.sync_copy(midx.at[wid], idx_v)
        waits = [
            pltpu.async_copy(mtab.at[idx_v.at[j]], rows_v.at[pl.ds(j * CHUNK, CHUNK)], sem)
            for j in range(NCH)
        ]
        for w in waits:
            w.wait()
        pltpu.sync_copy(rows_v, x.at[pl.ds(base, BPW), pl.ds(F, F)])

    return _sc_gather


BM = 1024


_NT = (((1,), (1,)), ((), ()))


def _dot_nt(a, w):
    return lax.dot_general(a, w, _NT, preferred_element_type=jnp.float32)


def _mlp_body(x, w1, b1, w2, b2, w3, b3, wf, bf, out):
    bf16 = jnp.bfloat16
    h = _dot_nt(x[...].astype(bf16), w1[...])
    h = jnp.maximum(h + b1[...], 0.0).astype(bf16)
    h = jnp.maximum(_dot_nt(h, w2[...]) + b2[...], 0.0).astype(bf16)
    h = jnp.maximum(_dot_nt(h, w3[...]) + b3[...], 0.0).astype(bf16)
    z = _dot_nt(wf[...].astype(bf16), h) + bf[0, 0]
    out[...] = (4.5 * jax.nn.sigmoid(z) + 0.5).reshape(BM // 128, 128)


def _mlp(x, w1, b1, w2, b2, w3, b3, wf, bf, interpret=False):
    const = lambda i: (0, 0)
    return pl.pallas_call(
        _mlp_body,
        grid=(BSUB // BM,),
        in_specs=[
            pl.BlockSpec((BM, 2 * F), lambda i: (i, 0)),
            pl.BlockSpec((H1, 2 * F), const),
            pl.BlockSpec((1, H1), const),
            pl.BlockSpec((H2, H1), const),
            pl.BlockSpec((1, H2), const),
            pl.BlockSpec((H3, H2), const),
            pl.BlockSpec((1, H3), const),
            pl.BlockSpec((1, H3), const),
            pl.BlockSpec((1, 1), const),
        ],
        out_specs=pl.BlockSpec((BM // 128, 128), lambda i: (i, 0)),
        out_shape=jax.ShapeDtypeStruct((BSUB // 128, 128), jnp.float32),
        interpret=interpret,
    )(x, w1, b1, w2, b2, w3, b3, wf, bf)


def kernel(users, movies, user_table, movie_table, W1, b1, W2, b2, W3, b3, Wf, bf):
    uidx = users.reshape(NSPLIT, NW, NCH, CHUNK)
    midx = movies.reshape(NSPLIT, NW, NCH, CHUNK)
    w1 = W1.astype(jnp.bfloat16)
    w2 = W2.astype(jnp.bfloat16)
    w3 = W3.astype(jnp.bfloat16)
    b1r, b2r, b3r = b1.reshape(1, H1), b2.reshape(1, H2), b3.reshape(1, H3)
    bfr = bf.reshape(1, 1)
    gather = _make_sc_gather()
    outs = []
    for c in range(NSPLIT):
        x = gather(uidx[c], midx[c], user_table, movie_table)
        outs.append(_mlp(x, w1, b1r, w2, b2r, w3, b3r, Wf, bfr))
    return jnp.concatenate(outs, axis=0).reshape(B, 1)


# speedup vs baseline: 1.2544x; 1.0300x over previous
import functools

import jax
import jax.numpy as jnp
from jax import lax
from jax.experimental import pallas as pl
from jax.experimental.pallas import tpu as pltpu
from jax.experimental.pallas import tpu_sc as plsc

B = 16384
F = 128
H1, H2, H3 = 1024, 512, 256

NC, NS = 2, 16
NW = NC * NS
CHUNK = 128
NSPLIT = 1
BSUB = B // NSPLIT
BPW = BSUB // NW
NCH = BPW // CHUNK

@functools.cache
def _make_sc_gather():
    mesh = plsc.VectorSubcoreMesh(core_axis_name="c", subcore_axis_name="s")

    @functools.partial(
        pl.kernel,
        mesh=mesh,
        out_type=jax.ShapeDtypeStruct((BSUB, 2 * F), jnp.float32),
        scratch_types=[
            pltpu.VMEM((NCH, CHUNK), jnp.int32),
            pltpu.VMEM((BPW, F), jnp.float32),
            pltpu.SemaphoreType.DMA,
        ],
    )
    def _sc_gather(uidx, midx, utab, mtab, x, idx_v, rows_v, sem):
        wid = lax.axis_index("s") * NC + lax.axis_index("c")
        base = wid * BPW
        pltpu.sync_copy(uidx.at[wid], idx_v)
        waits = [
            pltpu.async_copy(utab.at[idx_v.at[j]], rows_v.at[pl.ds(j * CHUNK, CHUNK)], sem)
            for j in range(NCH)
        ]
        for w in waits:
            w.wait()
        pltpu.sync_copy(rows_v, x.at[pl.ds(base, BPW), pl.ds(0, F)])
        pltpu.sync_copy(midx.at[wid], idx_v)
        waits = [
            pltpu.async_copy(mtab.at[idx_v.at[j]], rows_v.at[pl.ds(j * CHUNK, CHUNK)], sem)
            for j in range(NCH)
        ]
        for w in waits:
            w.wait()
        pltpu.sync_copy(rows_v, x.at[pl.ds(base, BPW), pl.ds(F, F)])

    return _sc_gather


BM = 2048


_NT = (((1,), (1,)), ((), ()))


def _dot_nt(a, w):
    return lax.dot_general(a, w, _NT, preferred_element_type=jnp.float32)


def _mlp_body(x, w1, b1, w2, b2, w3, b3, wf, bf, out):
    bf16 = jnp.bfloat16
    h = _dot_nt(x[...].astype(bf16), w1[...])
    h = jnp.maximum(h + b1[...], 0.0).astype(bf16)
    h = jnp.maximum(_dot_nt(h, w2[...]) + b2[...], 0.0).astype(bf16)
    h = jnp.maximum(_dot_nt(h, w3[...]) + b3[...], 0.0).astype(bf16)
    z = _dot_nt(wf[...].astype(bf16), h) + bf[0, 0]
    out[...] = (4.5 * jax.nn.sigmoid(z) + 0.5).reshape(BM // 128, 128)


def _mlp(x, w1, b1, w2, b2, w3, b3, wf, bf, interpret=False):
    const = lambda i: (0, 0)
    return pl.pallas_call(
        _mlp_body,
        grid=(BSUB // BM,),
        in_specs=[
            pl.BlockSpec((BM, 2 * F), lambda i: (i, 0)),
            pl.BlockSpec((H1, 2 * F), const),
            pl.BlockSpec((1, H1), const),
            pl.BlockSpec((H2, H1), const),
            pl.BlockSpec((1, H2), const),
            pl.BlockSpec((H3, H2), const),
            pl.BlockSpec((1, H3), const),
            pl.BlockSpec((1, H3), const),
            pl.BlockSpec((1, 1), const),
        ],
        out_specs=pl.BlockSpec((BM // 128, 128), lambda i: (i, 0)),
        out_shape=jax.ShapeDtypeStruct((BSUB // 128, 128), jnp.float32),
        interpret=interpret,
    )(x, w1, b1, w2, b2, w3, b3, wf, bf)


def kernel(users, movies, user_table, movie_table, W1, b1, W2, b2, W3, b3, Wf, bf):
    uidx = users.reshape(NSPLIT, NW, NCH, CHUNK)
    midx = movies.reshape(NSPLIT, NW, NCH, CHUNK)
    w1 = W1.astype(jnp.bfloat16)
    w2 = W2.astype(jnp.bfloat16)
    w3 = W3.astype(jnp.bfloat16)
    b1r, b2r, b3r = b1.reshape(1, H1), b2.reshape(1, H2), b3.reshape(1, H3)
    bfr = bf.reshape(1, 1)
    gather = _make_sc_gather()
    outs = []
    for c in range(NSPLIT):
        x = gather(uidx[c], midx[c], user_table, movie_table)
        outs.append(_mlp(x, w1, b1r, w2, b2r, w3, b3r, Wf, bfr))
    return jnp.concatenate(outs, axis=0).reshape(B, 1)


# speedup vs baseline: 1.2792x; 1.0198x over previous
import functools

import jax
import jax.numpy as jnp
from jax import lax
from jax.experimental import pallas as pl
from jax.experimental.pallas import tpu as pltpu
from jax.experimental.pallas import tpu_sc as plsc

B = 16384
F = 128
H1, H2, H3 = 1024, 512, 256

NC, NS = 2, 16
NW = NC * NS
CHUNK = 128
NSPLIT = 1
BSUB = B // NSPLIT
BPW = BSUB // NW
NCH = BPW // CHUNK

@functools.cache
def _make_sc_gather():
    mesh = plsc.VectorSubcoreMesh(core_axis_name="c", subcore_axis_name="s")

    @functools.partial(
        pl.kernel,
        mesh=mesh,
        out_type=jax.ShapeDtypeStruct((BSUB, 2 * F), jnp.float32),
        scratch_types=[
            pltpu.VMEM((NCH, CHUNK), jnp.int32),
            pltpu.VMEM((BPW, F), jnp.float32),
            pltpu.SemaphoreType.DMA,
        ],
    )
    def _sc_gather(uidx, midx, utab, mtab, x, idx_v, rows_v, sem):
        wid = lax.axis_index("s") * NC + lax.axis_index("c")
        base = wid * BPW
        pltpu.sync_copy(uidx.at[wid], idx_v)
        waits = [
            pltpu.async_copy(utab.at[idx_v.at[j]], rows_v.at[pl.ds(j * CHUNK, CHUNK)], sem)
            for j in range(NCH)
        ]
        for w in waits:
            w.wait()
        pltpu.sync_copy(rows_v, x.at[pl.ds(base, BPW), pl.ds(0, F)])
        pltpu.sync_copy(midx.at[wid], idx_v)
        waits = [
            pltpu.async_copy(mtab.at[idx_v.at[j]], rows_v.at[pl.ds(j * CHUNK, CHUNK)], sem)
            for j in range(NCH)
        ]
        for w in waits:
            w.wait()
        pltpu.sync_copy(rows_v, x.at[pl.ds(base, BPW), pl.ds(F, F)])

    return _sc_gather


BM = 4096


_NT = (((1,), (1,)), ((), ()))


def _dot_nt(a, w):
    return lax.dot_general(a, w, _NT, preferred_element_type=jnp.float32)


def _mlp_body(x, w1, b1, w2, b2, w3, b3, wf, bf, out):
    bf16 = jnp.bfloat16
    h = _dot_nt(x[...].astype(bf16), w1[...])
    h = jnp.maximum(h + b1[...], 0.0).astype(bf16)
    h = jnp.maximum(_dot_nt(h, w2[...]) + b2[...], 0.0).astype(bf16)
    h = jnp.maximum(_dot_nt(h, w3[...]) + b3[...], 0.0).astype(bf16)
    z = _dot_nt(wf[...].astype(bf16), h) + bf[0, 0]
    out[...] = (4.5 * jax.nn.sigmoid(z) + 0.5).reshape(BM // 128, 128)


def _mlp(x, w1, b1, w2, b2, w3, b3, wf, bf, interpret=False):
    const = lambda i: (0, 0)
    return pl.pallas_call(
        _mlp_body,
        grid=(BSUB // BM,),
        in_specs=[
            pl.BlockSpec((BM, 2 * F), lambda i: (i, 0)),
            pl.BlockSpec((H1, 2 * F), const),
            pl.BlockSpec((1, H1), const),
            pl.BlockSpec((H2, H1), const),
            pl.BlockSpec((1, H2), const),
            pl.BlockSpec((H3, H2), const),
            pl.BlockSpec((1, H3), const),
            pl.BlockSpec((1, H3), const),
            pl.BlockSpec((1, 1), const),
        ],
        out_specs=pl.BlockSpec((BM // 128, 128), lambda i: (i, 0)),
        out_shape=jax.ShapeDtypeStruct((BSUB // 128, 128), jnp.float32),
        interpret=interpret,
    )(x, w1, b1, w2, b2, w3, b3, wf, bf)


def kernel(users, movies, user_table, movie_table, W1, b1, W2, b2, W3, b3, Wf, bf):
    uidx = users.reshape(NSPLIT, NW, NCH, CHUNK)
    midx = movies.reshape(NSPLIT, NW, NCH, CHUNK)
    w1 = W1.astype(jnp.bfloat16)
    w2 = W2.astype(jnp.bfloat16)
    w3 = W3.astype(jnp.bfloat16)
    b1r, b2r, b3r = b1.reshape(1, H1), b2.reshape(1, H2), b3.reshape(1, H3)
    bfr = bf.reshape(1, 1)
    gather = _make_sc_gather()
    outs = []
    for c in range(NSPLIT):
        x = gather(uidx[c], midx[c], user_table, movie_table)
        outs.append(_mlp(x, w1, b1r, w2, b2r, w3, b3r, Wf, bfr))
    return jnp.concatenate(outs, axis=0).reshape(B, 1)
